# Initial kernel scaffold; baseline (speedup 1.0000x reference)
import jax
import jax.numpy as jnp
from jax.experimental import pallas as pl


def kernel(story, C0, C1, C2, C3):
    raise NotImplementedError("write your pallas kernel here")


# speedup vs baseline: 12.5366x; 12.5366x over previous
import functools

import jax
import jax.numpy as jnp
from jax import lax
from jax.experimental import pallas as pl
from jax.experimental.pallas import tpu as pltpu
from jax.experimental.pallas import tpu_sc as plsc

VOCAB = 100000
D = 64
N_TOK = 204800

NC, NS = 2, 16
CHUNKS, CW = 50, 128
TOK_PER_TILE = CHUNKS * CW
VPAD = 100096
SLICE = VPAD // NS

RB = 2000
NB = VOCAB // RB


def _hist_body(story_hbm, out_hbm, idx_v, ones_v, zer_v, counts_sp):
    c = lax.axis_index("c")
    s = lax.axis_index("s")

    def fill_ones(k, _):
        ones_v[pl.ds(k * 16, 16)] = jnp.full((16,), 1.0, jnp.float32)
        return _

    lax.fori_loop(0, CW // 16, fill_ones, None)

    def fill_zeros(k, _):
        zer_v[pl.ds(k * 16, 16)] = jnp.zeros((16,), jnp.float32)
        return _

    lax.fori_loop(0, SLICE // 16, fill_zeros, None)

    pltpu.sync_copy(zer_v, counts_sp.at[pl.ds(s * SLICE, SLICE)])
    pltpu.sync_copy(story_hbm.at[c * NS + s], idx_v)
    plsc.subcore_barrier()

    def scatter(j, _):
        pltpu.sync_copy(ones_v, counts_sp.at[idx_v.at[j]], add=True)
        return _

    lax.fori_loop(0, CHUNKS, scatter, None)
    plsc.subcore_barrier()

    pltpu.sync_copy(counts_sp.at[pl.ds(s * SLICE, SLICE)], zer_v)
    pltpu.sync_copy(zer_v, out_hbm.at[pl.ds(c * VPAD + s * SLICE, SLICE)])


@functools.cache
def _histogram():
    return pl.kernel(
        _hist_body,
        out_type=jax.ShapeDtypeStruct((NC * VPAD,), jnp.float32),
        mesh=plsc.VectorSubcoreMesh(
            core_axis_name="c", subcore_axis_name="s",
            num_cores=NC, num_subcores=NS,
        ),
        scratch_types=[
            pltpu.VMEM((CHUNKS, CW), jnp.int32),
            pltpu.VMEM((CW,), jnp.float32),
            pltpu.VMEM((SLICE,), jnp.float32),
            pltpu.VMEM_SHARED((VPAD,), jnp.float32),
        ],
    )


def _hops_body(cnt0, cnt1, c1, c2, c3, out, acc, u, m, z, o):
    p = pl.program_id(0)
    i = pl.program_id(1)
    cnt = cnt0[0] + cnt1[0]

    @pl.when((p == 0) & (i == 0))
    def _():
        acc[...] = jnp.zeros_like(acc)

    @pl.when(p == 0)
    def _():
        acc[...] += jnp.dot(cnt, c1[...], preferred_element_type=jnp.float32)

    @pl.when((p == 1) & (i == 0))
    def _():
        u[...] = acc[...] * (1.0 / N_TOK)
        m[...] = jnp.full_like(m, -1e30)
        z[...] = jnp.zeros_like(z)
        o[...] = jnp.zeros_like(o)

    @pl.when((p == 2) & (i == 0))
    def _():
        u[...] = u[...] + o[...] / z[...]
        m[...] = jnp.full_like(m, -1e30)
        z[...] = jnp.zeros_like(z)
        o[...] = jnp.zeros_like(o)

    @pl.when(p >= 1)
    def _():
        a_blk = jnp.where(p == 1, c1[...], c2[...])
        v_blk = jnp.where(p == 1, c2[...], c3[...])
        t = lax.dot_general(
            u[...], a_blk, (((1,), (1,)), ((), ())),
            preferred_element_type=jnp.float32,
        )
        m_old = m[0, 0]
        m_new = jnp.maximum(m_old, jnp.max(t))
        alpha = jnp.exp(m_old - m_new)
        w = cnt * jnp.exp(t - m_new)
        z[...] = z[...] * alpha + jnp.sum(w)
        o[...] = o[...] * alpha + jnp.dot(
            w, v_blk, preferred_element_type=jnp.float32
        )
        m[...] = jnp.full_like(m, m_new)

    @pl.when((p == 2) & (i == NB - 1))
    def _():
        out[...] = u[...] + o[...] / z[...]


def _hops(cnt0, cnt1, C1, C2, C3):
    return pl.pallas_call(
        _hops_body,
        grid=(3, NB),
        in_specs=[
            pl.BlockSpec((1, 1, RB), lambda p, i: (i, 0, 0)),
            pl.BlockSpec((1, 1, RB), lambda p, i: (i, 0, 0)),
            pl.BlockSpec((RB, D), lambda p, i: (jnp.where(p <= 1, i, NB - 1), 0)),
            pl.BlockSpec((RB, D), lambda p, i: (jnp.where(p >= 1, i, 0), 0)),
            pl.BlockSpec((RB, D), lambda p, i: (jnp.where(p == 2, i, 0), 0)),
        ],
        out_specs=pl.BlockSpec((1, D), lambda p, i: (0, 0)),
        out_shape=jax.ShapeDtypeStruct((1, D), jnp.float32),
        scratch_shapes=[
            pltpu.VMEM((1, D), jnp.float32),
            pltpu.VMEM((1, D), jnp.float32),
            pltpu.VMEM((1, 1), jnp.float32),
            pltpu.VMEM((1, 1), jnp.float32),
            pltpu.VMEM((1, D), jnp.float32),
        ],
        compiler_params=pltpu.CompilerParams(
            dimension_semantics=("arbitrary", "arbitrary"),
        ),
    )(cnt0, cnt1, C1, C2, C3)


@jax.jit
def kernel(story, C0, C1, C2, C3):
    del C0
    story_r = story.reshape(NC * NS, CHUNKS, CW)
    counts2 = _histogram()(story_r)
    cnt = counts2.reshape(NC, VPAD)[:, :VOCAB].reshape(NC, NB, 1, RB)
    return _hops(cnt[0], cnt[1], C1, C2, C3)


# speedup vs baseline: 16.8678x; 1.3455x over previous
import functools

import jax
import jax.numpy as jnp
from jax import lax
from jax.experimental import pallas as pl
from jax.experimental.pallas import tpu as pltpu
from jax.experimental.pallas import tpu_sc as plsc

VOCAB = 100000
D = 64
N_TOK = 204800

NC, NS = 2, 16
CHUNKS, CW = 50, 128
TOK_PER_TILE = CHUNKS * CW
VPAD = 100096
SLICE = VPAD // NS

RB = 10000
NB = VOCAB // RB


def _hist_body(story_hbm, out_hbm, idx_v, ones_v, zer_v, counts_sp):
    c = lax.axis_index("c")
    s = lax.axis_index("s")

    def fill_ones(k, _):
        ones_v[pl.ds(k * 16, 16)] = jnp.full((16,), 1.0, jnp.float32)
        return _

    lax.fori_loop(0, CW // 16, fill_ones, None)

    def fill_zeros(k, _):
        zer_v[pl.ds(k * 16, 16)] = jnp.zeros((16,), jnp.float32)
        return _

    lax.fori_loop(0, SLICE // 16, fill_zeros, None)

    pltpu.sync_copy(zer_v, counts_sp.at[pl.ds(s * SLICE, SLICE)])
    pltpu.sync_copy(story_hbm.at[c * NS + s], idx_v)
    plsc.subcore_barrier()

    def scatter(j, _):
        pltpu.sync_copy(ones_v, counts_sp.at[idx_v.at[j]], add=True)
        return _

    lax.fori_loop(0, CHUNKS, scatter, None)
    plsc.subcore_barrier()

    pltpu.sync_copy(counts_sp.at[pl.ds(s * SLICE, SLICE)], zer_v)
    pltpu.sync_copy(zer_v, out_hbm.at[pl.ds(c * VPAD + s * SLICE, SLICE)])


@functools.cache
def _histogram():
    return pl.kernel(
        _hist_body,
        out_type=jax.ShapeDtypeStruct((NC * VPAD,), jnp.float32),
        mesh=plsc.VectorSubcoreMesh(
            core_axis_name="c", subcore_axis_name="s",
            num_cores=NC, num_subcores=NS,
        ),
        scratch_types=[
            pltpu.VMEM((CHUNKS, CW), jnp.int32),
            pltpu.VMEM((CW,), jnp.float32),
            pltpu.VMEM((SLICE,), jnp.float32),
            pltpu.VMEM_SHARED((VPAD,), jnp.float32),
        ],
    )


def _hops_body(cnt0, cnt1, c1, c2, c3, out, acc, u, m, z, o):
    p = pl.program_id(0)
    i = pl.program_id(1)
    cnt = cnt0[0] + cnt1[0]

    @pl.when((p == 0) & (i == 0))
    def _():
        acc[...] = jnp.zeros_like(acc)

    @pl.when(p == 0)
    def _():
        acc[...] += jnp.dot(cnt, c1[...], preferred_element_type=jnp.float32)

    @pl.when((p == 1) & (i == 0))
    def _():
        u[...] = acc[...] * (1.0 / N_TOK)
        m[...] = jnp.full_like(m, -1e30)
        z[...] = jnp.zeros_like(z)
        o[...] = jnp.zeros_like(o)

    @pl.when((p == 2) & (i == 0))
    def _():
        u[...] = u[...] + o[...] / z[...]
        m[...] = jnp.full_like(m, -1e30)
        z[...] = jnp.zeros_like(z)
        o[...] = jnp.zeros_like(o)

    @pl.when(p >= 1)
    def _():
        a_blk = jnp.where(p == 1, c1[...], c2[...])
        v_blk = jnp.where(p == 1, c2[...], c3[...])
        t = lax.dot_general(
            u[...], a_blk, (((1,), (1,)), ((), ())),
            preferred_element_type=jnp.float32,
        )
        m_old = m[0, 0]
        m_new = jnp.maximum(m_old, jnp.max(t))
        alpha = jnp.exp(m_old - m_new)
        w = cnt * jnp.exp(t - m_new)
        z[...] = z[...] * alpha + jnp.sum(w)
        o[...] = o[...] * alpha + jnp.dot(
            w, v_blk, preferred_element_type=jnp.float32
        )
        m[...] = jnp.full_like(m, m_new)

    @pl.when((p == 2) & (i == NB - 1))
    def _():
        out[...] = u[...] + o[...] / z[...]


def _hops(cnt0, cnt1, C1, C2, C3):
    return pl.pallas_call(
        _hops_body,
        grid=(3, NB),
        in_specs=[
            pl.BlockSpec((1, 1, RB), lambda p, i: (i, 0, 0)),
            pl.BlockSpec((1, 1, RB), lambda p, i: (i, 0, 0)),
            pl.BlockSpec((RB, D), lambda p, i: (jnp.where(p <= 1, i, NB - 1), 0)),
            pl.BlockSpec((RB, D), lambda p, i: (jnp.where(p >= 1, i, 0), 0)),
            pl.BlockSpec((RB, D), lambda p, i: (jnp.where(p == 2, i, 0), 0)),
        ],
        out_specs=pl.BlockSpec((1, D), lambda p, i: (0, 0)),
        out_shape=jax.ShapeDtypeStruct((1, D), jnp.float32),
        scratch_shapes=[
            pltpu.VMEM((1, D), jnp.float32),
            pltpu.VMEM((1, D), jnp.float32),
            pltpu.VMEM((1, 1), jnp.float32),
            pltpu.VMEM((1, 1), jnp.float32),
            pltpu.VMEM((1, D), jnp.float32),
        ],
        compiler_params=pltpu.CompilerParams(
            dimension_semantics=("arbitrary", "arbitrary"),
        ),
    )(cnt0, cnt1, C1, C2, C3)


@jax.jit
def kernel(story, C0, C1, C2, C3):
    del C0
    story_r = story.reshape(NC * NS, CHUNKS, CW)
    counts2 = _histogram()(story_r)
    cnt = counts2.reshape(NC, VPAD)[:, :VOCAB].reshape(NC, NB, 1, RB)
    return _hops(cnt[0], cnt[1], C1, C2, C3)


# speedup vs baseline: 16.8736x; 1.0003x over previous
import functools

import jax
import jax.numpy as jnp
from jax import lax
from jax.experimental import pallas as pl
from jax.experimental.pallas import tpu as pltpu
from jax.experimental.pallas import tpu_sc as plsc

VOCAB = 100000
D = 64
N_TOK = 204800

NC, NS = 2, 16
CHUNKS, CW = 50, 128
TOK_PER_TILE = CHUNKS * CW
VPAD = 100096
SLICE = VPAD // NS

RB = 10000
NB = VOCAB // RB


def _hist_body(story_hbm, out_hbm, idx_v, ones_v, zer_v, counts_sp, sem):
    c = lax.axis_index("c")
    s = lax.axis_index("s")

    def fill_ones(k, _):
        ones_v[k // 8, pl.ds((k % 8) * 16, 16)] = jnp.full((16,), 1.0, jnp.float32)
        return _

    lax.fori_loop(0, CHUNKS * CW // 16, fill_ones, None)

    def fill_zeros(k, _):
        zer_v[pl.ds(k * 16, 16)] = jnp.zeros((16,), jnp.float32)
        return _

    lax.fori_loop(0, SLICE // 16, fill_zeros, None)

    pltpu.sync_copy(zer_v, counts_sp.at[pl.ds(s * SLICE, SLICE)])
    pltpu.sync_copy(story_hbm.at[c * NS + s], idx_v)
    plsc.subcore_barrier()

    def scatter_start(j, _):
        pltpu.async_copy(ones_v.at[j], counts_sp.at[idx_v.at[j]], sem, add=True)
        return _

    lax.fori_loop(0, CHUNKS, scatter_start, None)

    def scatter_wait(j, _):
        pltpu.make_async_copy(ones_v.at[j], counts_sp.at[idx_v.at[j]], sem).wait()
        return _

    lax.fori_loop(0, CHUNKS, scatter_wait, None)
    plsc.subcore_barrier()

    pltpu.sync_copy(counts_sp.at[pl.ds(s * SLICE, SLICE)], zer_v)
    pltpu.sync_copy(zer_v, out_hbm.at[pl.ds(c * VPAD + s * SLICE, SLICE)])


@functools.cache
def _histogram():
    return pl.kernel(
        _hist_body,
        out_type=jax.ShapeDtypeStruct((NC * VPAD,), jnp.float32),
        mesh=plsc.VectorSubcoreMesh(
            core_axis_name="c", subcore_axis_name="s",
            num_cores=NC, num_subcores=NS,
        ),
        scratch_types=[
            pltpu.VMEM((CHUNKS, CW), jnp.int32),
            pltpu.VMEM((CHUNKS, CW), jnp.float32),
            pltpu.VMEM((SLICE,), jnp.float32),
            pltpu.VMEM_SHARED((VPAD,), jnp.float32),
            pltpu.SemaphoreType.DMA,
        ],
    )


def _hops_body(cnt0, cnt1, c1, c2, c3, out, acc, u, m, z, o):
    p = pl.program_id(0)
    i = pl.program_id(1)
    cnt = cnt0[0] + cnt1[0]

    @pl.when((p == 0) & (i == 0))
    def _():
        acc[...] = jnp.zeros_like(acc)

    @pl.when(p == 0)
    def _():
        acc[...] += jnp.dot(cnt, c1[...], preferred_element_type=jnp.float32)

    @pl.when((p == 1) & (i == 0))
    def _():
        u[...] = acc[...] * (1.0 / N_TOK)
        m[...] = jnp.full_like(m, -1e30)
        z[...] = jnp.zeros_like(z)
        o[...] = jnp.zeros_like(o)

    @pl.when((p == 2) & (i == 0))
    def _():
        u[...] = u[...] + o[...] / z[...]
        m[...] = jnp.full_like(m, -1e30)
        z[...] = jnp.zeros_like(z)
        o[...] = jnp.zeros_like(o)

    @pl.when(p >= 1)
    def _():
        a_blk = jnp.where(p == 1, c1[...], c2[...])
        v_blk = jnp.where(p == 1, c2[...], c3[...])
        t = lax.dot_general(
            u[...], a_blk, (((1,), (1,)), ((), ())),
            preferred_element_type=jnp.float32,
        )
        m_old = m[0, 0]
        m_new = jnp.maximum(m_old, jnp.max(t))
        alpha = jnp.exp(m_old - m_new)
        w = cnt * jnp.exp(t - m_new)
        z[...] = z[...] * alpha + jnp.sum(w)
        o[...] = o[...] * alpha + jnp.dot(
            w, v_blk, preferred_element_type=jnp.float32
        )
        m[...] = jnp.full_like(m, m_new)

    @pl.when((p == 2) & (i == NB - 1))
    def _():
        out[...] = u[...] + o[...] / z[...]


def _hops(cnt0, cnt1, C1, C2, C3):
    return pl.pallas_call(
        _hops_body,
        grid=(3, NB),
        in_specs=[
            pl.BlockSpec((1, 1, RB), lambda p, i: (i, 0, 0)),
            pl.BlockSpec((1, 1, RB), lambda p, i: (i, 0, 0)),
            pl.BlockSpec((RB, D), lambda p, i: (jnp.where(p <= 1, i, NB - 1), 0)),
            pl.BlockSpec((RB, D), lambda p, i: (jnp.where(p >= 1, i, 0), 0)),
            pl.BlockSpec((RB, D), lambda p, i: (jnp.where(p == 2, i, 0), 0)),
        ],
        out_specs=pl.BlockSpec((1, D), lambda p, i: (0, 0)),
        out_shape=jax.ShapeDtypeStruct((1, D), jnp.float32),
        scratch_shapes=[
            pltpu.VMEM((1, D), jnp.float32),
            pltpu.VMEM((1, D), jnp.float32),
            pltpu.VMEM((1, 1), jnp.float32),
            pltpu.VMEM((1, 1), jnp.float32),
            pltpu.VMEM((1, D), jnp.float32),
        ],
        compiler_params=pltpu.CompilerParams(
            dimension_semantics=("arbitrary", "arbitrary"),
        ),
    )(cnt0, cnt1, C1, C2, C3)


@jax.jit
def kernel(story, C0, C1, C2, C3):
    del C0
    story_r = story.reshape(NC * NS, CHUNKS, CW)
    counts2 = _histogram()(story_r)
    cnt = counts2.reshape(NC, VPAD)[:, :VOCAB].reshape(NC, NB, 1, RB)
    return _hops(cnt[0], cnt[1], C1, C2, C3)
